# Initial kernel scaffold; baseline (speedup 1.0000x reference)
import functools

import jax
import jax.numpy as jnp
from jax import lax
from jax.experimental import pallas as pl
from jax.experimental.pallas import tpu as pltpu
from jax.experimental.pallas import tpu_sc as plsc

N = 10000
E = 320000
D = 128
F = 32

NC = 2
NS = 16
NW = NC * NS

K = 80
EPT = E // NW
CPT = EPT // K
RPT = N // NS

_MESH = plsc.VectorSubcoreMesh(
    core_axis_name="c", subcore_axis_name="s", num_cores=NC, num_subcores=NS)


@functools.partial(
    pl.kernel,
    out_type=jax.ShapeDtypeStruct((NC, N, 8), jnp.float32),
    mesh=_MESH,
    scratch_types=[
        pltpu.VMEM_SHARED((N, 8), jnp.float32),
        pltpu.VMEM((CPT, K), jnp.int32),
        pltpu.VMEM((K, 8), jnp.float32),
    ],
)
def _sc_degree(dst_hbm, ones_hbm, zeros_hbm, out_hbm, acc, idx_v, ones_v):
    c = lax.axis_index("c")
    s = lax.axis_index("s")
    wid = c * NS + s
    pltpu.sync_copy(zeros_hbm, acc.at[pl.ds(s * RPT, RPT)])
    pltpu.sync_copy(dst_hbm.at[pl.ds(wid * CPT, CPT)], idx_v)
    pltpu.sync_copy(ones_hbm, ones_v)
    plsc.subcore_barrier()

    @pl.loop(0, CPT)
    def _(j):
        pltpu.sync_copy(ones_v, acc.at[idx_v.at[j]], add=True)

    plsc.subcore_barrier()
    pltpu.sync_copy(acc.at[pl.ds(s * RPT, RPT)],
                    out_hbm.at[c, pl.ds(s * RPT, RPT)])


@functools.partial(
    pl.kernel,
    out_type=jax.ShapeDtypeStruct((NC, N, F), jnp.float32),
    mesh=_MESH,
    scratch_types=[
        pltpu.VMEM_SHARED((N, F), jnp.float32),
        pltpu.VMEM((CPT, K), jnp.int32),
        pltpu.VMEM((CPT, K), jnp.int32),
        pltpu.VMEM((K, F), jnp.float32),
        pltpu.SemaphoreType.DMA,
    ],
)
def _sc_propagate(g_hbm, src_hbm, dst_hbm, zeros_hbm, out_hbm,
                  acc, src_v, dst_v, rows_v, sem):
    c = lax.axis_index("c")
    s = lax.axis_index("s")
    wid = c * NS + s
    pltpu.sync_copy(zeros_hbm, acc.at[pl.ds(s * RPT, RPT)])
    pltpu.sync_copy(src_hbm.at[pl.ds(wid * CPT, CPT)], src_v)
    pltpu.sync_copy(dst_hbm.at[pl.ds(wid * CPT, CPT)], dst_v)
    plsc.subcore_barrier()

    @pl.loop(0, CPT)
    def _(j):
        pltpu.async_copy(g_hbm.at[src_v.at[j]], rows_v, sem).wait()
        pltpu.sync_copy(rows_v, acc.at[dst_v.at[j]], add=True)

    plsc.subcore_barrier()
    pltpu.sync_copy(acc.at[pl.ds(s * RPT, RPT)],
                    out_hbm.at[c, pl.ds(s * RPT, RPT)])


R = 400
G = N // R


def _degs(pA_ref):
    deg = pA_ref[0, :, 0] + pA_ref[1, :, 0] + 1.0
    inv = lax.rsqrt(deg)
    return deg, inv


def _tc_pre_body(pA_ref, x_ref, w1_ref, hw_ref, g_ref):
    _, inv = _degs(pA_ref)
    hw = jnp.dot(x_ref[...], w1_ref[...], preferred_element_type=jnp.float32)
    hw_ref[...] = hw
    g_ref[...] = hw * inv[:, None]


def _tc_mid_body(pA_ref, p_ref, hw_ref, b_ref, w2_ref, hw2_ref, g2_ref):
    deg, inv = _degs(pA_ref)
    agg = p_ref[0] + p_ref[1]
    h = inv[:, None] * agg + hw_ref[...] * (1.0 / deg)[:, None] + b_ref[...]
    h = jnp.where(h > 0, h, jnp.expm1(h))
    hw2 = jnp.dot(h, w2_ref[...], preferred_element_type=jnp.float32)
    hw2_ref[...] = hw2
    g2_ref[...] = hw2 * inv[:, None]


def _tc_post_body(pA_ref, p_ref, hw_ref, b_ref, wf1_ref, bf1_ref,
                  wf2_ref, bf2_ref, out_ref, acc_ref):
    i = pl.program_id(0)
    deg, inv = _degs(pA_ref)
    agg = p_ref[0] + p_ref[1]
    h = inv[:, None] * agg + hw_ref[...] * (1.0 / deg)[:, None] + b_ref[...]
    h = jnp.where(h > 0, h, jnp.expm1(h))
    part = jnp.sum(h, axis=0, keepdims=True)

    @pl.when(i == 0)
    def _():
        acc_ref[...] = jnp.zeros_like(acc_ref)

    acc_ref[...] += part

    @pl.when(i == G - 1)
    def _():
        pooled = acc_ref[...]
        f = jnp.dot(pooled, wf1_ref[...],
                    preferred_element_type=jnp.float32) + bf1_ref[...]
        f = jnp.maximum(f, 0.0)
        o = jnp.dot(f, wf2_ref[...],
                    preferred_element_type=jnp.float32) + bf2_ref[...]
        out_ref[...] = 1.0 / (1.0 + jnp.exp(-o))


def _row_spec(w):
    return pl.BlockSpec((R, w), lambda i: (i, 0))


def _pA_spec():
    return pl.BlockSpec((NC, R, 8), lambda i: (0, i, 0))


def _p_spec():
    return pl.BlockSpec((NC, R, F), lambda i: (0, i, 0))


def _full_spec(shape):
    return pl.BlockSpec(shape, lambda i: tuple(0 for _ in shape))


_tc_pre = pl.pallas_call(
    _tc_pre_body,
    grid=(G,),
    in_specs=[_pA_spec(), _row_spec(D), _full_spec((D, F))],
    out_specs=[_row_spec(F), _row_spec(F)],
    out_shape=[jax.ShapeDtypeStruct((N, F), jnp.float32),
               jax.ShapeDtypeStruct((N, F), jnp.float32)],
)

_tc_mid = pl.pallas_call(
    _tc_mid_body,
    grid=(G,),
    in_specs=[_pA_spec(), _p_spec(), _row_spec(F), _full_spec((1, F)),
              _full_spec((F, F))],
    out_specs=[_row_spec(F), _row_spec(F)],
    out_shape=[jax.ShapeDtypeStruct((N, F), jnp.float32),
               jax.ShapeDtypeStruct((N, F), jnp.float32)],
)

_tc_post = pl.pallas_call(
    _tc_post_body,
    grid=(G,),
    in_specs=[_pA_spec(), _p_spec(), _row_spec(F), _full_spec((1, F)),
              _full_spec((F, 512)), _full_spec((1, 512)),
              _full_spec((512, 1)), _full_spec((1, 1))],
    out_specs=_full_spec((1, 1)),
    out_shape=jax.ShapeDtypeStruct((1, 1), jnp.float32),
    scratch_shapes=[pltpu.VMEM((1, F), jnp.float32)],
)


def kernel(x, W1, b1, W2, b2, Wf1, bf1, Wf2, bf2, edge_index):
    src = edge_index[0].reshape(E // K, K)
    dst = edge_index[1].reshape(E // K, K)
    ones8 = jnp.ones((K, 8), jnp.float32)
    zeros8 = jnp.zeros((RPT, 8), jnp.float32)
    zerosF = jnp.zeros((RPT, F), jnp.float32)

    pA = _sc_degree(dst, ones8, zeros8)
    hw1, g1 = _tc_pre(pA, x, W1)
    p1 = _sc_propagate(g1, src, dst, zerosF)
    hw2, g2 = _tc_mid(pA, p1, hw1, b1.reshape(1, F), W2)
    p2 = _sc_propagate(g2, src, dst, zerosF)
    out = _tc_post(pA, p2, hw2, b2.reshape(1, F), Wf1, bf1.reshape(1, 512),
                   Wf2, bf2.reshape(1, 1))
    return out


# speedup vs baseline: 21.3402x; 21.3402x over previous
import functools

import jax
import jax.numpy as jnp
from jax import lax
from jax.experimental import pallas as pl
from jax.experimental.pallas import tpu as pltpu
from jax.experimental.pallas import tpu_sc as plsc

N = 10000
E = 320000
D = 128
F = 32

NC = 2
NS = 16
NW = NC * NS

K = 80
EPT = E // NW
CPT = EPT // K
NP = 10240
RPT = NP // NS

_MESH = plsc.VectorSubcoreMesh(
    core_axis_name="c", subcore_axis_name="s", num_cores=NC, num_subcores=NS)


@functools.partial(
    pl.kernel,
    out_type=jax.ShapeDtypeStruct((NC, NP, 8), jnp.float32),
    mesh=_MESH,
    scratch_types=[
        pltpu.VMEM_SHARED((NP, 8), jnp.float32),
        pltpu.VMEM((CPT, K), jnp.int32),
        pltpu.VMEM((K, 8), jnp.float32),
    ],
    compiler_params=pltpu.CompilerParams(use_tc_tiling_on_sc=False),
)
def _sc_degree(dst_hbm, ones_hbm, zeros_hbm, out_hbm, acc, idx_v, ones_v):
    c = lax.axis_index("c")
    s = lax.axis_index("s")
    wid = c * NS + s
    pltpu.sync_copy(zeros_hbm, acc.at[pl.ds(s * RPT, RPT)])
    pltpu.sync_copy(dst_hbm.at[wid], idx_v)
    pltpu.sync_copy(ones_hbm, ones_v)
    plsc.subcore_barrier()

    @pl.loop(0, CPT)
    def _(j):
        pltpu.sync_copy(ones_v, acc.at[idx_v.at[j]], add=True)

    plsc.subcore_barrier()
    pltpu.sync_copy(acc.at[pl.ds(s * RPT, RPT)],
                    out_hbm.at[c, pl.ds(s * RPT, RPT)])


@functools.partial(
    pl.kernel,
    out_type=jax.ShapeDtypeStruct((NC, NP, F), jnp.float32),
    mesh=_MESH,
    scratch_types=[
        pltpu.VMEM_SHARED((NP, F), jnp.float32),
        pltpu.VMEM((CPT, K), jnp.int32),
        pltpu.VMEM((CPT, K), jnp.int32),
        pltpu.VMEM((K, F), jnp.float32),
        pltpu.SemaphoreType.DMA,
    ],
    compiler_params=pltpu.CompilerParams(use_tc_tiling_on_sc=False),
)
def _sc_propagate(g_hbm, src_hbm, dst_hbm, zeros_hbm, out_hbm,
                  acc, src_v, dst_v, rows_v, sem):
    c = lax.axis_index("c")
    s = lax.axis_index("s")
    wid = c * NS + s
    pltpu.sync_copy(zeros_hbm, acc.at[pl.ds(s * RPT, RPT)])
    pltpu.sync_copy(src_hbm.at[wid], src_v)
    pltpu.sync_copy(dst_hbm.at[wid], dst_v)
    plsc.subcore_barrier()

    @pl.loop(0, CPT)
    def _(j):
        pltpu.async_copy(g_hbm.at[src_v.at[j]], rows_v, sem).wait()
        pltpu.sync_copy(rows_v, acc.at[dst_v.at[j]], add=True)

    plsc.subcore_barrier()
    pltpu.sync_copy(acc.at[pl.ds(s * RPT, RPT)],
                    out_hbm.at[c, pl.ds(s * RPT, RPT)])


R = 400
G = N // R


def _degs(pA_ref):
    deg = pA_ref[0, :, 0] + pA_ref[1, :, 0] + 1.0
    inv = lax.rsqrt(deg)
    return deg, inv


def _tc_pre_body(pA_ref, x_ref, w1_ref, hw_ref, g_ref):
    _, inv = _degs(pA_ref)
    hw = jnp.dot(x_ref[...], w1_ref[...], preferred_element_type=jnp.float32)
    hw_ref[...] = hw
    g_ref[...] = hw * inv[:, None]


def _tc_mid_body(pA_ref, p_ref, hw_ref, b_ref, w2_ref, hw2_ref, g2_ref):
    deg, inv = _degs(pA_ref)
    agg = p_ref[0] + p_ref[1]
    h = inv[:, None] * agg + hw_ref[...] * (1.0 / deg)[:, None] + b_ref[...]
    h = jnp.where(h > 0, h, jnp.exp(h) - 1.0)
    hw2 = jnp.dot(h, w2_ref[...], preferred_element_type=jnp.float32)
    hw2_ref[...] = hw2
    g2_ref[...] = hw2 * inv[:, None]


def _tc_post_body(pA_ref, p_ref, hw_ref, b_ref, wf1_ref, bf1_ref,
                  wf2_ref, bf2_ref, out_ref, acc_ref):
    i = pl.program_id(0)
    deg, inv = _degs(pA_ref)
    agg = p_ref[0] + p_ref[1]
    h = inv[:, None] * agg + hw_ref[...] * (1.0 / deg)[:, None] + b_ref[...]
    h = jnp.where(h > 0, h, jnp.exp(h) - 1.0)
    part = jnp.sum(h, axis=0, keepdims=True)

    @pl.when(i == 0)
    def _():
        acc_ref[...] = jnp.zeros_like(acc_ref)

    acc_ref[...] += part

    @pl.when(i == G - 1)
    def _():
        pooled = acc_ref[...]
        f = jnp.dot(pooled, wf1_ref[...],
                    preferred_element_type=jnp.float32) + bf1_ref[...]
        f = jnp.maximum(f, 0.0)
        o = jnp.dot(f, wf2_ref[...],
                    preferred_element_type=jnp.float32) + bf2_ref[...]
        out_ref[...] = 1.0 / (1.0 + jnp.exp(-o))


def _row_spec(w):
    return pl.BlockSpec((R, w), lambda i: (i, 0))


def _pA_spec():
    return pl.BlockSpec((NC, R, 8), lambda i: (0, i, 0))


def _p_spec():
    return pl.BlockSpec((NC, R, F), lambda i: (0, i, 0))


def _full_spec(shape):
    return pl.BlockSpec(shape, lambda i: tuple(0 for _ in shape))


_tc_pre = pl.pallas_call(
    _tc_pre_body,
    grid=(G,),
    in_specs=[_pA_spec(), _row_spec(D), _full_spec((D, F))],
    out_specs=[_row_spec(F), _row_spec(F)],
    out_shape=[jax.ShapeDtypeStruct((N, F), jnp.float32),
               jax.ShapeDtypeStruct((N, F), jnp.float32)],
)

_tc_mid = pl.pallas_call(
    _tc_mid_body,
    grid=(G,),
    in_specs=[_pA_spec(), _p_spec(), _row_spec(F), _full_spec((1, F)),
              _full_spec((F, F))],
    out_specs=[_row_spec(F), _row_spec(F)],
    out_shape=[jax.ShapeDtypeStruct((N, F), jnp.float32),
               jax.ShapeDtypeStruct((N, F), jnp.float32)],
)

_tc_post = pl.pallas_call(
    _tc_post_body,
    grid=(G,),
    in_specs=[_pA_spec(), _p_spec(), _row_spec(F), _full_spec((1, F)),
              _full_spec((F, 512)), _full_spec((1, 512)),
              _full_spec((512, 1)), _full_spec((1, 1))],
    out_specs=_full_spec((1, 1)),
    out_shape=jax.ShapeDtypeStruct((1, 1), jnp.float32),
    scratch_shapes=[pltpu.VMEM((1, F), jnp.float32)],
)


def kernel(x, W1, b1, W2, b2, Wf1, bf1, Wf2, bf2, edge_index):
    src = edge_index[0].reshape(NW, CPT, K)
    dst = edge_index[1].reshape(NW, CPT, K)
    ones8 = jnp.ones((K, 8), jnp.float32)
    zeros8 = jnp.zeros((RPT, 8), jnp.float32)
    zerosF = jnp.zeros((RPT, F), jnp.float32)

    pA = _sc_degree(dst, ones8, zeros8)
    hw1, g1 = _tc_pre(pA, x, W1)
    p1 = _sc_propagate(g1, src, dst, zerosF)
    hw2, g2 = _tc_mid(pA, p1, hw1, b1.reshape(1, F), W2)
    p2 = _sc_propagate(g2, src, dst, zerosF)
    out = _tc_post(pA, p2, hw2, b2.reshape(1, F), Wf1, bf1.reshape(1, 512),
                   Wf2, bf2.reshape(1, 1))
    return out


# speedup vs baseline: 37.2219x; 1.7442x over previous
import functools

import jax
import jax.numpy as jnp
from jax import lax
from jax.experimental import pallas as pl
from jax.experimental.pallas import tpu as pltpu
from jax.experimental.pallas import tpu_sc as plsc

N = 10000
E = 320000
D = 128
F = 32

NC = 2
NS = 16
NW = NC * NS

K = 80
EPT = E // NW
CPT = EPT // K
NP = 10240
RPT = NP // NS

_MESH = plsc.VectorSubcoreMesh(
    core_axis_name="c", subcore_axis_name="s", num_cores=NC, num_subcores=NS)


@functools.partial(
    pl.kernel,
    out_type=jax.ShapeDtypeStruct((NC, NP, 8), jnp.float32),
    mesh=_MESH,
    scratch_types=[
        pltpu.VMEM_SHARED((NP, 8), jnp.float32),
        pltpu.VMEM((CPT, K), jnp.int32),
        pltpu.VMEM((K, 8), jnp.float32),
    ],
    compiler_params=pltpu.CompilerParams(use_tc_tiling_on_sc=False),
)
def _sc_degree(dst_hbm, ones_hbm, zeros_hbm, out_hbm, acc, idx_v, ones_v):
    c = lax.axis_index("c")
    s = lax.axis_index("s")
    wid = c * NS + s
    pltpu.sync_copy(zeros_hbm, acc.at[pl.ds(s * RPT, RPT)])
    pltpu.sync_copy(dst_hbm.at[wid], idx_v)
    pltpu.sync_copy(ones_hbm, ones_v)
    plsc.subcore_barrier()

    @pl.loop(0, CPT)
    def _(j):
        pltpu.sync_copy(ones_v, acc.at[idx_v.at[j]], add=True)

    plsc.subcore_barrier()
    pltpu.sync_copy(acc.at[pl.ds(s * RPT, RPT)],
                    out_hbm.at[c, pl.ds(s * RPT, RPT)])


GS = 5
NB = 2 * GS
NR = CPT // GS


@functools.partial(
    pl.kernel,
    out_type=jax.ShapeDtypeStruct((NC, NP, F), jnp.float32),
    mesh=_MESH,
    scratch_types=[
        pltpu.VMEM_SHARED((NP, F), jnp.float32),
        pltpu.VMEM((CPT, K), jnp.int32),
        pltpu.VMEM((CPT, K), jnp.int32),
        pltpu.VMEM((NB, K, F), jnp.float32),
        pltpu.SemaphoreType.DMA((NB,)),
        pltpu.SemaphoreType.DMA((NB,)),
    ],
    compiler_params=pltpu.CompilerParams(use_tc_tiling_on_sc=False),
)
def _sc_propagate(g_hbm, src_hbm, dst_hbm, zeros_hbm, out_hbm,
                  acc, src_v, dst_v, rows_v, sem_g, sem_s):
    c = lax.axis_index("c")
    s = lax.axis_index("s")
    wid = c * NS + s
    pltpu.sync_copy(zeros_hbm, acc.at[pl.ds(s * RPT, RPT)])
    pltpu.sync_copy(src_hbm.at[wid], src_v)
    pltpu.sync_copy(dst_hbm.at[wid], dst_v)

    def fire_gathers(j0, half):
        for i in range(GS):
            b = half * GS + i
            pltpu.async_copy(g_hbm.at[src_v.at[j0 + i]], rows_v.at[b],
                             sem_g.at[b])

    def wait_gathers(half):
        for i in range(GS):
            b = half * GS + i
            pltpu.make_async_copy(g_hbm.at[src_v.at[0]], rows_v.at[b],
                                  sem_g.at[b]).wait()

    def fire_scatters(j0, half):
        for i in range(GS):
            b = half * GS + i
            pltpu.async_copy(rows_v.at[b], acc.at[dst_v.at[j0 + i]],
                             sem_s.at[b], add=True)

    def wait_scatters(half):
        for i in range(GS):
            b = half * GS + i
            pltpu.make_async_copy(rows_v.at[b], acc.at[dst_v.at[0]],
                                  sem_s.at[b]).wait()

    fire_gathers(0, 0)
    plsc.subcore_barrier()

    @pl.loop(0, NR, step=2)
    def _(r0):
        @pl.when(r0 + 1 < NR)
        def _():
            @pl.when(r0 >= 2)
            def _():
                wait_scatters(1)
            fire_gathers((r0 + 1) * GS, 1)
        wait_gathers(0)
        fire_scatters(r0 * GS, 0)

        @pl.when(r0 + 1 < NR)
        def _():
            @pl.when(r0 + 2 < NR)
            def _():
                wait_scatters(0)
                fire_gathers((r0 + 2) * GS, 0)
            wait_gathers(1)
            fire_scatters((r0 + 1) * GS, 1)

    wait_scatters(0)
    wait_scatters(1)

    plsc.subcore_barrier()
    pltpu.sync_copy(acc.at[pl.ds(s * RPT, RPT)],
                    out_hbm.at[c, pl.ds(s * RPT, RPT)])


R = 400
G = N // R


def _degs(pA_ref):
    deg = pA_ref[0, :, 0] + pA_ref[1, :, 0] + 1.0
    inv = lax.rsqrt(deg)
    return deg, inv


def _tc_pre_body(pA_ref, x_ref, w1_ref, hw_ref, g_ref):
    _, inv = _degs(pA_ref)
    hw = jnp.dot(x_ref[...], w1_ref[...], preferred_element_type=jnp.float32)
    hw_ref[...] = hw
    g_ref[...] = hw * inv[:, None]


def _tc_mid_body(pA_ref, p_ref, hw_ref, b_ref, w2_ref, hw2_ref, g2_ref):
    deg, inv = _degs(pA_ref)
    agg = p_ref[0] + p_ref[1]
    h = inv[:, None] * agg + hw_ref[...] * (1.0 / deg)[:, None] + b_ref[...]
    h = jnp.where(h > 0, h, jnp.exp(h) - 1.0)
    hw2 = jnp.dot(h, w2_ref[...], preferred_element_type=jnp.float32)
    hw2_ref[...] = hw2
    g2_ref[...] = hw2 * inv[:, None]


def _tc_post_body(pA_ref, p_ref, hw_ref, b_ref, wf1_ref, bf1_ref,
                  wf2_ref, bf2_ref, out_ref, acc_ref):
    i = pl.program_id(0)
    deg, inv = _degs(pA_ref)
    agg = p_ref[0] + p_ref[1]
    h = inv[:, None] * agg + hw_ref[...] * (1.0 / deg)[:, None] + b_ref[...]
    h = jnp.where(h > 0, h, jnp.exp(h) - 1.0)
    part = jnp.sum(h, axis=0, keepdims=True)

    @pl.when(i == 0)
    def _():
        acc_ref[...] = jnp.zeros_like(acc_ref)

    acc_ref[...] += part

    @pl.when(i == G - 1)
    def _():
        pooled = acc_ref[...]
        f = jnp.dot(pooled, wf1_ref[...],
                    preferred_element_type=jnp.float32) + bf1_ref[...]
        f = jnp.maximum(f, 0.0)
        o = jnp.dot(f, wf2_ref[...],
                    preferred_element_type=jnp.float32) + bf2_ref[...]
        out_ref[...] = 1.0 / (1.0 + jnp.exp(-o))


def _row_spec(w):
    return pl.BlockSpec((R, w), lambda i: (i, 0))


def _pA_spec():
    return pl.BlockSpec((NC, R, 8), lambda i: (0, i, 0))


def _p_spec():
    return pl.BlockSpec((NC, R, F), lambda i: (0, i, 0))


def _full_spec(shape):
    return pl.BlockSpec(shape, lambda i: tuple(0 for _ in shape))


_tc_pre = pl.pallas_call(
    _tc_pre_body,
    grid=(G,),
    in_specs=[_pA_spec(), _row_spec(D), _full_spec((D, F))],
    out_specs=[_row_spec(F), _row_spec(F)],
    out_shape=[jax.ShapeDtypeStruct((N, F), jnp.float32),
               jax.ShapeDtypeStruct((N, F), jnp.float32)],
)

_tc_mid = pl.pallas_call(
    _tc_mid_body,
    grid=(G,),
    in_specs=[_pA_spec(), _p_spec(), _row_spec(F), _full_spec((1, F)),
              _full_spec((F, F))],
    out_specs=[_row_spec(F), _row_spec(F)],
    out_shape=[jax.ShapeDtypeStruct((N, F), jnp.float32),
               jax.ShapeDtypeStruct((N, F), jnp.float32)],
)

_tc_post = pl.pallas_call(
    _tc_post_body,
    grid=(G,),
    in_specs=[_pA_spec(), _p_spec(), _row_spec(F), _full_spec((1, F)),
              _full_spec((F, 512)), _full_spec((1, 512)),
              _full_spec((512, 1)), _full_spec((1, 1))],
    out_specs=_full_spec((1, 1)),
    out_shape=jax.ShapeDtypeStruct((1, 1), jnp.float32),
    scratch_shapes=[pltpu.VMEM((1, F), jnp.float32)],
)


def kernel(x, W1, b1, W2, b2, Wf1, bf1, Wf2, bf2, edge_index):
    src = edge_index[0].reshape(NW, CPT, K)
    dst = edge_index[1].reshape(NW, CPT, K)
    ones8 = jnp.ones((K, 8), jnp.float32)
    zeros8 = jnp.zeros((RPT, 8), jnp.float32)
    zerosF = jnp.zeros((RPT, F), jnp.float32)

    pA = _sc_degree(dst, ones8, zeros8)
    hw1, g1 = _tc_pre(pA, x, W1)
    p1 = _sc_propagate(g1, src, dst, zerosF)
    hw2, g2 = _tc_mid(pA, p1, hw1, b1.reshape(1, F), W2)
    p2 = _sc_propagate(g2, src, dst, zerosF)
    out = _tc_post(pA, p2, hw2, b2.reshape(1, F), Wf1, bf1.reshape(1, 512),
                   Wf2, bf2.reshape(1, 1))
    return out


# speedup vs baseline: 38.8050x; 1.0425x over previous
import functools

import jax
import jax.numpy as jnp
from jax import lax
from jax.experimental import pallas as pl
from jax.experimental.pallas import tpu as pltpu
from jax.experimental.pallas import tpu_sc as plsc

N = 10000
E = 320000
D = 128
F = 32

NC = 2
NS = 16
NW = NC * NS

K = 80
EPT = E // NW
CPT = EPT // K
NP = 10240
RPT = NP // NS

_MESH = plsc.VectorSubcoreMesh(
    core_axis_name="c", subcore_axis_name="s", num_cores=NC, num_subcores=NS)


@functools.partial(
    pl.kernel,
    out_type=jax.ShapeDtypeStruct((NC, NP, 8), jnp.float32),
    mesh=_MESH,
    scratch_types=[
        pltpu.VMEM_SHARED((NP, 8), jnp.float32),
        pltpu.VMEM((CPT, K), jnp.int32),
        pltpu.VMEM((K, 8), jnp.float32),
        pltpu.SemaphoreType.DMA,
    ],
    compiler_params=pltpu.CompilerParams(use_tc_tiling_on_sc=False),
)
def _sc_degree(dst_hbm, ones_hbm, zeros_hbm, out_hbm, acc, idx_v, ones_v, sem):
    c = lax.axis_index("c")
    s = lax.axis_index("s")
    wid = c * NS + s
    pltpu.sync_copy(zeros_hbm, acc.at[pl.ds(s * RPT, RPT)])
    pltpu.sync_copy(dst_hbm.at[wid], idx_v)
    pltpu.sync_copy(ones_hbm, ones_v)
    plsc.subcore_barrier()

    DW = 24

    @pl.loop(0, DW)
    def _(j):
        pltpu.async_copy(ones_v, acc.at[idx_v.at[j]], sem, add=True)

    @pl.loop(DW, CPT)
    def _(j):
        pltpu.make_async_copy(ones_v, acc.at[idx_v.at[0]], sem).wait()
        pltpu.async_copy(ones_v, acc.at[idx_v.at[j]], sem, add=True)

    @pl.loop(0, DW)
    def _(j):
        pltpu.make_async_copy(ones_v, acc.at[idx_v.at[0]], sem).wait()

    plsc.subcore_barrier()
    pltpu.sync_copy(acc.at[pl.ds(s * RPT, RPT)],
                    out_hbm.at[c, pl.ds(s * RPT, RPT)])


GS = 5
NGRP = 5
NB = NGRP * GS
NR = CPT // GS
PF = 2


@functools.partial(
    pl.kernel,
    out_type=jax.ShapeDtypeStruct((NC, NP, F), jnp.float32),
    mesh=_MESH,
    scratch_types=[
        pltpu.VMEM_SHARED((NP, F), jnp.float32),
        pltpu.VMEM((CPT, K), jnp.int32),
        pltpu.VMEM((CPT, K), jnp.int32),
        pltpu.VMEM((NB, K, F), jnp.float32),
        pltpu.SemaphoreType.DMA((NGRP,)),
        pltpu.SemaphoreType.DMA((NGRP,)),
    ],
    compiler_params=pltpu.CompilerParams(use_tc_tiling_on_sc=False),
)
def _sc_propagate(g_hbm, src_hbm, dst_hbm, zeros_hbm, out_hbm,
                  acc, src_v, dst_v, rows_v, sem_g, sem_s):
    c = lax.axis_index("c")
    s = lax.axis_index("s")
    wid = c * NS + s
    pltpu.sync_copy(zeros_hbm, acc.at[pl.ds(s * RPT, RPT)])
    pltpu.sync_copy(src_hbm.at[wid], src_v)
    pltpu.sync_copy(dst_hbm.at[wid], dst_v)

    def fire_gathers(j0, grp):
        for i in range(GS):
            b = grp * GS + i
            pltpu.async_copy(g_hbm.at[src_v.at[j0 + i]], rows_v.at[b],
                             sem_g.at[grp])

    def wait_gathers(grp):
        for i in range(GS):
            b = grp * GS + i
            pltpu.make_async_copy(g_hbm.at[src_v.at[0]], rows_v.at[b],
                                  sem_g.at[grp]).wait()

    def fire_scatters(j0, grp):
        for i in range(GS):
            b = grp * GS + i
            pltpu.async_copy(rows_v.at[b], acc.at[dst_v.at[j0 + i]],
                             sem_s.at[grp], add=True)

    def wait_scatters(grp):
        for i in range(GS):
            b = grp * GS + i
            pltpu.make_async_copy(rows_v.at[b], acc.at[dst_v.at[0]],
                                  sem_s.at[grp]).wait()

    for r in range(PF):
        fire_gathers(r * GS, r)
    plsc.subcore_barrier()

    @pl.loop(0, NR, step=NGRP)
    def _(r0):
        for i in range(NGRP):
            r = r0 + i
            wait_gathers(i)
            fire_scatters(r * GS, i)
            gp = (i + PF) % NGRP

            @pl.when(r + PF < NR)
            def _():
                @pl.when(r + PF >= NGRP)
                def _():
                    wait_scatters(gp)
                fire_gathers((r + PF) * GS, gp)

    for r in range(NR - NGRP, NR):
        wait_scatters(r % NGRP)

    plsc.subcore_barrier()
    pltpu.sync_copy(acc.at[pl.ds(s * RPT, RPT)],
                    out_hbm.at[c, pl.ds(s * RPT, RPT)])


R = 400
G = N // R


def _degs(pA_ref):
    deg = pA_ref[0, :, 0] + pA_ref[1, :, 0] + 1.0
    inv = lax.rsqrt(deg)
    return deg, inv


def _tc_pre_body(pA_ref, x_ref, w1_ref, hw_ref, g_ref):
    _, inv = _degs(pA_ref)
    hw = jnp.dot(x_ref[...], w1_ref[...], preferred_element_type=jnp.float32)
    hw_ref[...] = hw
    g_ref[...] = hw * inv[:, None]


def _tc_mid_body(pA_ref, p_ref, hw_ref, b_ref, w2_ref, hw2_ref, g2_ref):
    deg, inv = _degs(pA_ref)
    agg = p_ref[0] + p_ref[1]
    h = inv[:, None] * agg + hw_ref[...] * (1.0 / deg)[:, None] + b_ref[...]
    h = jnp.where(h > 0, h, jnp.exp(h) - 1.0)
    hw2 = jnp.dot(h, w2_ref[...], preferred_element_type=jnp.float32)
    hw2_ref[...] = hw2
    g2_ref[...] = hw2 * inv[:, None]


def _tc_post_body(pA_ref, p_ref, hw_ref, b_ref, wf1_ref, bf1_ref,
                  wf2_ref, bf2_ref, out_ref, acc_ref):
    i = pl.program_id(0)
    deg, inv = _degs(pA_ref)
    agg = p_ref[0] + p_ref[1]
    h = inv[:, None] * agg + hw_ref[...] * (1.0 / deg)[:, None] + b_ref[...]
    h = jnp.where(h > 0, h, jnp.exp(h) - 1.0)
    part = jnp.sum(h, axis=0, keepdims=True)

    @pl.when(i == 0)
    def _():
        acc_ref[...] = jnp.zeros_like(acc_ref)

    acc_ref[...] += part

    @pl.when(i == G - 1)
    def _():
        pooled = acc_ref[...]
        f = jnp.dot(pooled, wf1_ref[...],
                    preferred_element_type=jnp.float32) + bf1_ref[...]
        f = jnp.maximum(f, 0.0)
        o = jnp.dot(f, wf2_ref[...],
                    preferred_element_type=jnp.float32) + bf2_ref[...]
        out_ref[...] = 1.0 / (1.0 + jnp.exp(-o))


def _row_spec(w):
    return pl.BlockSpec((R, w), lambda i: (i, 0))


def _pA_spec():
    return pl.BlockSpec((NC, R, 8), lambda i: (0, i, 0))


def _p_spec():
    return pl.BlockSpec((NC, R, F), lambda i: (0, i, 0))


def _full_spec(shape):
    return pl.BlockSpec(shape, lambda i: tuple(0 for _ in shape))


_tc_pre = pl.pallas_call(
    _tc_pre_body,
    grid=(G,),
    in_specs=[_pA_spec(), _row_spec(D), _full_spec((D, F))],
    out_specs=[_row_spec(F), _row_spec(F)],
    out_shape=[jax.ShapeDtypeStruct((N, F), jnp.float32),
               jax.ShapeDtypeStruct((N, F), jnp.float32)],
)

_tc_mid = pl.pallas_call(
    _tc_mid_body,
    grid=(G,),
    in_specs=[_pA_spec(), _p_spec(), _row_spec(F), _full_spec((1, F)),
              _full_spec((F, F))],
    out_specs=[_row_spec(F), _row_spec(F)],
    out_shape=[jax.ShapeDtypeStruct((N, F), jnp.float32),
               jax.ShapeDtypeStruct((N, F), jnp.float32)],
)

_tc_post = pl.pallas_call(
    _tc_post_body,
    grid=(G,),
    in_specs=[_pA_spec(), _p_spec(), _row_spec(F), _full_spec((1, F)),
              _full_spec((F, 512)), _full_spec((1, 512)),
              _full_spec((512, 1)), _full_spec((1, 1))],
    out_specs=_full_spec((1, 1)),
    out_shape=jax.ShapeDtypeStruct((1, 1), jnp.float32),
    scratch_shapes=[pltpu.VMEM((1, F), jnp.float32)],
)


def kernel(x, W1, b1, W2, b2, Wf1, bf1, Wf2, bf2, edge_index):
    src = edge_index[0].reshape(NW, CPT, K)
    dst = edge_index[1].reshape(NW, CPT, K)
    ones8 = jnp.ones((K, 8), jnp.float32)
    zeros8 = jnp.zeros((RPT, 8), jnp.float32)
    zerosF = jnp.zeros((RPT, F), jnp.float32)

    pA = _sc_degree(dst, ones8, zeros8)
    hw1, g1 = _tc_pre(pA, x, W1)
    p1 = _sc_propagate(g1, src, dst, zerosF)
    hw2, g2 = _tc_mid(pA, p1, hw1, b1.reshape(1, F), W2)
    p2 = _sc_propagate(g2, src, dst, zerosF)
    out = _tc_post(pA, p2, hw2, b2.reshape(1, F), Wf1, bf1.reshape(1, 512),
                   Wf2, bf2.reshape(1, 1))
    return out


# speedup vs baseline: 44.3511x; 1.1429x over previous
import functools

import jax
import jax.numpy as jnp
from jax import lax
from jax.experimental import pallas as pl
from jax.experimental.pallas import tpu as pltpu
from jax.experimental.pallas import tpu_sc as plsc

N = 10000
E = 320000
D = 128
F = 32

NC = 2
NS = 16
NW = NC * NS

K = 80
EPT = E // NW
CPT = EPT // K
NP = 10240
RPT = NP // NS

_MESH = plsc.VectorSubcoreMesh(
    core_axis_name="c", subcore_axis_name="s", num_cores=NC, num_subcores=NS)


@functools.partial(
    pl.kernel,
    out_type=jax.ShapeDtypeStruct((NC, NP, 8), jnp.float32),
    mesh=_MESH,
    scratch_types=[
        pltpu.VMEM_SHARED((NP, 8), jnp.float32),
        pltpu.VMEM((CPT, K), jnp.int32),
        pltpu.VMEM((K, 8), jnp.float32),
        pltpu.SemaphoreType.DMA,
    ],
    compiler_params=pltpu.CompilerParams(use_tc_tiling_on_sc=False),
)
def _sc_degree(dst_hbm, ones_hbm, zeros_hbm, out_hbm, acc, idx_v, ones_v, sem):
    c = lax.axis_index("c")
    s = lax.axis_index("s")
    wid = c * NS + s
    pltpu.sync_copy(zeros_hbm, acc.at[pl.ds(s * RPT, RPT)])
    pltpu.sync_copy(dst_hbm.at[wid], idx_v)
    pltpu.sync_copy(ones_hbm, ones_v)
    plsc.subcore_barrier()

    DW = 24

    @pl.loop(0, DW)
    def _(j):
        pltpu.async_copy(ones_v, acc.at[idx_v.at[j]], sem, add=True)

    @pl.loop(DW, CPT)
    def _(j):
        pltpu.make_async_copy(ones_v, acc.at[idx_v.at[0]], sem).wait()
        pltpu.async_copy(ones_v, acc.at[idx_v.at[j]], sem, add=True)

    @pl.loop(0, DW)
    def _(j):
        pltpu.make_async_copy(ones_v, acc.at[idx_v.at[0]], sem).wait()

    plsc.subcore_barrier()
    pltpu.sync_copy(acc.at[pl.ds(s * RPT, RPT)],
                    out_hbm.at[c, pl.ds(s * RPT, RPT)])


GS = 5
NGRP = 5
NB = NGRP * GS
NR = CPT // GS
PF = 2


@functools.partial(
    pl.kernel,
    out_type=jax.ShapeDtypeStruct((NC, NP, F), jnp.float32),
    mesh=_MESH,
    scratch_types=[
        pltpu.VMEM_SHARED((NP, F), jnp.float32),
        pltpu.VMEM((CPT, K), jnp.int32),
        pltpu.VMEM((CPT, K), jnp.int32),
        pltpu.VMEM((NB, K, F), jnp.float32),
        pltpu.SemaphoreType.DMA((NGRP,)),
        pltpu.SemaphoreType.DMA((NGRP,)),
    ],
    compiler_params=pltpu.CompilerParams(use_tc_tiling_on_sc=False),
)
def _sc_propagate(g_hbm, src_hbm, dst_hbm, zeros_hbm, out_hbm,
                  acc, src_v, dst_v, rows_v, sem_g, sem_s):
    c = lax.axis_index("c")
    s = lax.axis_index("s")
    wid = c * NS + s
    pltpu.sync_copy(zeros_hbm, acc.at[pl.ds(s * RPT, RPT)])
    pltpu.sync_copy(src_hbm.at[wid], src_v)
    pltpu.sync_copy(dst_hbm.at[wid], dst_v)

    def fire_gathers(j0, grp):
        for i in range(GS):
            b = grp * GS + i
            pltpu.async_copy(g_hbm.at[src_v.at[j0 + i]], rows_v.at[b],
                             sem_g.at[grp])

    def wait_gathers(grp):
        for i in range(GS):
            b = grp * GS + i
            pltpu.make_async_copy(g_hbm.at[src_v.at[0]], rows_v.at[b],
                                  sem_g.at[grp]).wait()

    def fire_scatters(j0, grp):
        for i in range(GS):
            b = grp * GS + i
            pltpu.async_copy(rows_v.at[b], acc.at[dst_v.at[j0 + i]],
                             sem_s.at[grp], add=True)

    def wait_scatters(grp):
        for i in range(GS):
            b = grp * GS + i
            pltpu.make_async_copy(rows_v.at[b], acc.at[dst_v.at[0]],
                                  sem_s.at[grp]).wait()

    for r in range(PF):
        fire_gathers(r * GS, r)
    plsc.subcore_barrier()

    @pl.loop(0, NR, step=NGRP)
    def _(r0):
        for i in range(NGRP):
            r = r0 + i
            wait_gathers(i)
            fire_scatters(r * GS, i)
            gp = (i + PF) % NGRP

            @pl.when(r + PF < NR)
            def _():
                @pl.when(r + PF >= NGRP)
                def _():
                    wait_scatters(gp)
                fire_gathers((r + PF) * GS, gp)

    for r in range(NR - NGRP, NR):
        wait_scatters(r % NGRP)

    plsc.subcore_barrier()
    pltpu.sync_copy(acc.at[pl.ds(s * RPT, RPT)],
                    out_hbm.at[c, pl.ds(s * RPT, RPT)])


def _degs(pA_ref):
    pA = pA_ref[...]
    deg = pA[0, :N, 0] + pA[1, :N, 0] + 1.0
    inv = lax.rsqrt(deg)
    return deg, inv


def _tc_pre_body(pA_ref, x_ref, w1_ref, hw_ref, g_ref):
    _, inv = _degs(pA_ref)
    hw = jnp.dot(x_ref[...], w1_ref[...], preferred_element_type=jnp.float32)
    hw_ref[...] = hw
    g_ref[...] = hw * inv[:, None]


def _combine(pA_ref, p_ref, hw_ref, b_ref):
    deg, inv = _degs(pA_ref)
    p = p_ref[...]
    agg = p[0, :N] + p[1, :N]
    h = inv[:, None] * agg + hw_ref[...] * (1.0 / deg)[:, None] + b_ref[...]
    return jnp.where(h > 0, h, jnp.exp(h) - 1.0), inv


def _tc_mid_body(pA_ref, p_ref, hw_ref, b_ref, w2_ref, hw2_ref, g2_ref):
    h, inv = _combine(pA_ref, p_ref, hw_ref, b_ref)
    hw2 = jnp.dot(h, w2_ref[...], preferred_element_type=jnp.float32)
    hw2_ref[...] = hw2
    g2_ref[...] = hw2 * inv[:, None]


def _tc_post_body(pA_ref, p_ref, hw_ref, b_ref, wf1_ref, bf1_ref,
                  wf2_ref, bf2_ref, out_ref):
    h, _ = _combine(pA_ref, p_ref, hw_ref, b_ref)
    pooled = jnp.sum(h, axis=0, keepdims=True)
    f = jnp.dot(pooled, wf1_ref[...],
                preferred_element_type=jnp.float32) + bf1_ref[...]
    f = jnp.maximum(f, 0.0)
    o = jnp.dot(f, wf2_ref[...],
                preferred_element_type=jnp.float32) + bf2_ref[...]
    out_ref[...] = 1.0 / (1.0 + jnp.exp(-o))


_tc_pre = pl.pallas_call(
    _tc_pre_body,
    out_shape=[jax.ShapeDtypeStruct((N, F), jnp.float32),
               jax.ShapeDtypeStruct((N, F), jnp.float32)],
)

_tc_mid = pl.pallas_call(
    _tc_mid_body,
    out_shape=[jax.ShapeDtypeStruct((N, F), jnp.float32),
               jax.ShapeDtypeStruct((N, F), jnp.float32)],
)

_tc_post = pl.pallas_call(
    _tc_post_body,
    out_shape=jax.ShapeDtypeStruct((1, 1), jnp.float32),
)


def kernel(x, W1, b1, W2, b2, Wf1, bf1, Wf2, bf2, edge_index):
    src = edge_index[0].reshape(NW, CPT, K)
    dst = edge_index[1].reshape(NW, CPT, K)
    ones8 = jnp.ones((K, 8), jnp.float32)
    zeros8 = jnp.zeros((RPT, 8), jnp.float32)
    zerosF = jnp.zeros((RPT, F), jnp.float32)

    pA = _sc_degree(dst, ones8, zeros8)
    hw1, g1 = _tc_pre(pA, x, W1)
    p1 = _sc_propagate(g1, src, dst, zerosF)
    hw2, g2 = _tc_mid(pA, p1, hw1, b1.reshape(1, F), W2)
    p2 = _sc_propagate(g2, src, dst, zerosF)
    out = _tc_post(pA, p2, hw2, b2.reshape(1, F), Wf1, bf1.reshape(1, 512),
                   Wf2, bf2.reshape(1, 1))
    return out


# speedup vs baseline: 58.7079x; 1.3237x over previous
import functools

import jax
import jax.numpy as jnp
from jax import lax
from jax.experimental import pallas as pl
from jax.experimental.pallas import tpu as pltpu
from jax.experimental.pallas import tpu_sc as plsc

N = 10000
E = 320000
D = 128
F = 32

NC = 2
NS = 16
NW = NC * NS

K = 80
EPT = E // NW
CPT = EPT // K
NP = 10240
RPT = NP // NS

_MESH = plsc.VectorSubcoreMesh(
    core_axis_name="c", subcore_axis_name="s", num_cores=NC, num_subcores=NS)


@functools.partial(
    pl.kernel,
    out_type=jax.ShapeDtypeStruct((NC, NP, F), jnp.float32),
    mesh=_MESH,
    scratch_types=[
        pltpu.VMEM_SHARED((NP, F), jnp.float32),
        pltpu.VMEM((CPT, K), jnp.int32),
        pltpu.VMEM((K, F), jnp.float32),
        pltpu.SemaphoreType.DMA,
    ],
    compiler_params=pltpu.CompilerParams(use_tc_tiling_on_sc=False),
)
def _sc_degree(dst_hbm, ones_hbm, zeros_hbm, out_hbm, acc, idx_v, ones_v, sem):
    c = lax.axis_index("c")
    s = lax.axis_index("s")
    wid = c * NS + s
    pltpu.sync_copy(zeros_hbm, acc.at[pl.ds(s * RPT, RPT)])
    pltpu.sync_copy(dst_hbm.at[wid], idx_v)
    pltpu.sync_copy(ones_hbm, ones_v)
    plsc.subcore_barrier()

    DW = 24

    @pl.loop(0, DW)
    def _(j):
        pltpu.async_copy(ones_v, acc.at[idx_v.at[j]], sem, add=True)

    @pl.loop(DW, CPT)
    def _(j):
        pltpu.make_async_copy(ones_v, acc.at[idx_v.at[0]], sem).wait()
        pltpu.async_copy(ones_v, acc.at[idx_v.at[j]], sem, add=True)

    @pl.loop(0, DW)
    def _(j):
        pltpu.make_async_copy(ones_v, acc.at[idx_v.at[0]], sem).wait()

    plsc.subcore_barrier()
    pltpu.sync_copy(acc.at[pl.ds(s * RPT, RPT)],
                    out_hbm.at[c, pl.ds(s * RPT, RPT)])


GS = 5
NGRP = 5
NB = NGRP * GS
NR = CPT // GS
PF = 2


@functools.partial(
    pl.kernel,
    out_type=jax.ShapeDtypeStruct((NC, NP, F), jnp.float32),
    mesh=_MESH,
    scratch_types=[
        pltpu.VMEM_SHARED((NP, F), jnp.float32),
        pltpu.VMEM((CPT, K), jnp.int32),
        pltpu.VMEM((CPT, K), jnp.int32),
        pltpu.VMEM((NB, K, F), jnp.float32),
        pltpu.SemaphoreType.DMA((NGRP,)),
        pltpu.SemaphoreType.DMA((NGRP,)),
    ],
    compiler_params=pltpu.CompilerParams(use_tc_tiling_on_sc=False),
)
def _sc_propagate(g_hbm, src_hbm, dst_hbm, zeros_hbm, out_hbm,
                  acc, src_v, dst_v, rows_v, sem_g, sem_s):
    c = lax.axis_index("c")
    s = lax.axis_index("s")
    wid = c * NS + s

    @pl.when(c == 0)
    def _():
        @pl.when(s < NS - 1)
        def _():
            pltpu.sync_copy(g_hbm.at[pl.ds(s * RPT, RPT)],
                            acc.at[pl.ds(s * RPT, RPT)])

        @pl.when(s == NS - 1)
        def _():
            last = N - (NS - 1) * RPT
            pltpu.sync_copy(g_hbm.at[pl.ds((NS - 1) * RPT, last)],
                            acc.at[pl.ds((NS - 1) * RPT, last)])
            pltpu.sync_copy(zeros_hbm.at[pl.ds(0, NP - N)],
                            acc.at[pl.ds(N, NP - N)])

    @pl.when(c == 1)
    def _():
        pltpu.sync_copy(zeros_hbm, acc.at[pl.ds(s * RPT, RPT)])

    pltpu.sync_copy(src_hbm.at[wid], src_v)
    pltpu.sync_copy(dst_hbm.at[wid], dst_v)

    def fire_gathers(j0, grp):
        for i in range(GS):
            b = grp * GS + i
            pltpu.async_copy(g_hbm.at[src_v.at[j0 + i]], rows_v.at[b],
                             sem_g.at[grp])

    def wait_gathers(grp):
        for i in range(GS):
            b = grp * GS + i
            pltpu.make_async_copy(g_hbm.at[src_v.at[0]], rows_v.at[b],
                                  sem_g.at[grp]).wait()

    def fire_scatters(j0, grp):
        for i in range(GS):
            b = grp * GS + i
            pltpu.async_copy(rows_v.at[b], acc.at[dst_v.at[j0 + i]],
                             sem_s.at[grp], add=True)

    def wait_scatters(grp):
        for i in range(GS):
            b = grp * GS + i
            pltpu.make_async_copy(rows_v.at[b], acc.at[dst_v.at[0]],
                                  sem_s.at[grp]).wait()

    for r in range(PF):
        fire_gathers(r * GS, r)
    plsc.subcore_barrier()

    @pl.loop(0, NR, step=NGRP)
    def _(r0):
        for i in range(NGRP):
            r = r0 + i
            wait_gathers(i)
            fire_scatters(r * GS, i)
            gp = (i + PF) % NGRP

            @pl.when(r + PF < NR)
            def _():
                @pl.when(r + PF >= NGRP)
                def _():
                    wait_scatters(gp)
                fire_gathers((r + PF) * GS, gp)

    for r in range(NR - NGRP, NR):
        wait_scatters(r % NGRP)

    plsc.subcore_barrier()
    pltpu.sync_copy(acc.at[pl.ds(s * RPT, RPT)],
                    out_hbm.at[c, pl.ds(s * RPT, RPT)])


NPK = N * F // 128
NPP = NP * F // 128


def _degs(pA_ref):
    pA = pA_ref[...]
    deg = pA[0, :NPK] + pA[1, :NPK] + 1.0
    inv = lax.rsqrt(deg)
    return deg, inv


def _tc_pre_body(pA_ref, x4_ref, w1bd_ref, g_ref):
    _, inv = _degs(pA_ref)
    hw = jnp.dot(x4_ref[...], w1bd_ref[...],
                 preferred_element_type=jnp.float32)
    g_ref[...] = hw * inv


def _combine(pA_ref, p_ref, b_ref):
    _, inv = _degs(pA_ref)
    p = p_ref[...]
    agg = p[0, :NPK] + p[1, :NPK]
    h = inv * agg + b_ref[...]
    return jnp.where(h > 0, h, jnp.exp(h) - 1.0), inv


def _tc_mid_body(pA_ref, p_ref, b_ref, w2bd_ref, g2_ref):
    h, inv = _combine(pA_ref, p_ref, b_ref)
    hw2 = jnp.dot(h, w2bd_ref[...], preferred_element_type=jnp.float32)
    g2_ref[...] = hw2 * inv


def _tc_post_body(pA_ref, p_ref, b_ref, wf1s_ref, bf1_ref,
                  wf2_ref, bf2_ref, out_ref):
    h, _ = _combine(pA_ref, p_ref, b_ref)
    pooled = jnp.sum(h, axis=0, keepdims=True)
    f = jnp.dot(pooled, wf1s_ref[...],
                preferred_element_type=jnp.float32) + bf1_ref[...]
    f = jnp.maximum(f, 0.0)
    o = jnp.dot(f, wf2_ref[...],
                preferred_element_type=jnp.float32) + bf2_ref[...]
    out_ref[...] = 1.0 / (1.0 + jnp.exp(-o))


_tc_pre = pl.pallas_call(
    _tc_pre_body,
    out_shape=jax.ShapeDtypeStruct((NPK, 128), jnp.float32),
)

_tc_mid = pl.pallas_call(
    _tc_mid_body,
    out_shape=jax.ShapeDtypeStruct((NPK, 128), jnp.float32),
)

_tc_post = pl.pallas_call(
    _tc_post_body,
    out_shape=jax.ShapeDtypeStruct((1, 1), jnp.float32),
)


def kernel(x, W1, b1, W2, b2, Wf1, bf1, Wf2, bf2, edge_index):
    src = edge_index[0].reshape(NW, CPT, K)
    dst = edge_index[1].reshape(NW, CPT, K)
    onesF = jnp.ones((K, F), jnp.float32)
    zerosF = jnp.zeros((RPT, F), jnp.float32)
    x4 = x.reshape(NPK, 4 * D)
    eye4 = jnp.eye(4, dtype=jnp.float32)
    W1bd = jnp.kron(eye4, W1)
    W2bd = jnp.kron(eye4, W2)
    Wf1s = jnp.concatenate([Wf1] * 4, axis=0)
    b1p = jnp.tile(b1, 4).reshape(1, 128)
    b2p = jnp.tile(b2, 4).reshape(1, 128)

    pA = _sc_degree(dst, onesF, zerosF).reshape(NC, NPP, 128)
    g1 = _tc_pre(pA, x4, W1bd)
    p1 = _sc_propagate(g1.reshape(N, F), src, dst, zerosF)
    g2 = _tc_mid(pA, p1.reshape(NC, NPP, 128), b1p, W2bd)
    p2 = _sc_propagate(g2.reshape(N, F), src, dst, zerosF)
    out = _tc_post(pA, p2.reshape(NC, NPP, 128), b2p, Wf1s,
                   bf1.reshape(1, 512), Wf2, bf2.reshape(1, 1))
    return out


# speedup vs baseline: 63.8181x; 1.0870x over previous
import functools

import jax
import jax.numpy as jnp
from jax import lax
from jax.experimental import pallas as pl
from jax.experimental.pallas import tpu as pltpu
from jax.experimental.pallas import tpu_sc as plsc

N = 10000
E = 320000
D = 128
F = 32

NC = 2
NS = 16
NW = NC * NS

K = 80
EPT = E // NW
CPT = EPT // K
NP = 10240
RPT = NP // NS

_MESH = plsc.VectorSubcoreMesh(
    core_axis_name="c", subcore_axis_name="s", num_cores=NC, num_subcores=NS)


@functools.partial(
    pl.kernel,
    out_type=jax.ShapeDtypeStruct((NC, NP, F), jnp.float32),
    mesh=_MESH,
    scratch_types=[
        pltpu.VMEM_SHARED((NP, F), jnp.float32),
        pltpu.VMEM((CPT, K), jnp.int32),
        pltpu.VMEM((K, F), jnp.float32),
        pltpu.SemaphoreType.DMA,
    ],
    compiler_params=pltpu.CompilerParams(use_tc_tiling_on_sc=False),
)
def _sc_degree(ei_hbm, ones_hbm, zeros_hbm, out_hbm, acc, idx_v, ones_v, sem):
    c = lax.axis_index("c")
    s = lax.axis_index("s")
    wid = c * NS + s
    pltpu.sync_copy(zeros_hbm, acc.at[pl.ds(s * RPT, RPT)])
    pltpu.sync_copy(ei_hbm.at[1, wid], idx_v)
    pltpu.sync_copy(ones_hbm, ones_v)
    plsc.subcore_barrier()

    DW = 24

    @pl.loop(0, DW)
    def _(j):
        pltpu.async_copy(ones_v, acc.at[idx_v.at[j]], sem, add=True)

    @pl.loop(DW, CPT)
    def _(j):
        pltpu.make_async_copy(ones_v, acc.at[idx_v.at[0]], sem).wait()
        pltpu.async_copy(ones_v, acc.at[idx_v.at[j]], sem, add=True)

    @pl.loop(0, DW)
    def _(j):
        pltpu.make_async_copy(ones_v, acc.at[idx_v.at[0]], sem).wait()

    plsc.subcore_barrier()
    pltpu.sync_copy(acc.at[pl.ds(s * RPT, RPT)],
                    out_hbm.at[c, pl.ds(s * RPT, RPT)])


GS = 5
NGRP = 5
NB = NGRP * GS
NR = CPT // GS
PF = 2


@functools.partial(
    pl.kernel,
    out_type=jax.ShapeDtypeStruct((NC, NP, F), jnp.float32),
    mesh=_MESH,
    scratch_types=[
        pltpu.VMEM_SHARED((NP, F), jnp.float32),
        pltpu.VMEM((CPT, K), jnp.int32),
        pltpu.VMEM((CPT, K), jnp.int32),
        pltpu.VMEM((NB, K, F), jnp.float32),
        pltpu.SemaphoreType.DMA((NGRP,)),
        pltpu.SemaphoreType.DMA((NGRP,)),
    ],
    compiler_params=pltpu.CompilerParams(use_tc_tiling_on_sc=False),
)
def _sc_propagate(g_hbm, ei_hbm, zeros_hbm, out_hbm,
                  acc, src_v, dst_v, rows_v, sem_g, sem_s):
    c = lax.axis_index("c")
    s = lax.axis_index("s")
    wid = c * NS + s

    @pl.when(c == 0)
    def _():
        @pl.when(s < NS - 1)
        def _():
            pltpu.sync_copy(g_hbm.at[pl.ds(s * RPT, RPT)],
                            acc.at[pl.ds(s * RPT, RPT)])

        @pl.when(s == NS - 1)
        def _():
            last = N - (NS - 1) * RPT
            pltpu.sync_copy(g_hbm.at[pl.ds((NS - 1) * RPT, last)],
                            acc.at[pl.ds((NS - 1) * RPT, last)])
            pltpu.sync_copy(zeros_hbm.at[pl.ds(0, NP - N)],
                            acc.at[pl.ds(N, NP - N)])

    @pl.when(c == 1)
    def _():
        pltpu.sync_copy(zeros_hbm, acc.at[pl.ds(s * RPT, RPT)])

    pltpu.sync_copy(ei_hbm.at[0, wid], src_v)
    pltpu.sync_copy(ei_hbm.at[1, wid], dst_v)

    def fire_gathers(j0, grp):
        for i in range(GS):
            b = grp * GS + i
            pltpu.async_copy(g_hbm.at[src_v.at[j0 + i]], rows_v.at[b],
                             sem_g.at[grp])

    def wait_gathers(grp):
        for i in range(GS):
            b = grp * GS + i
            pltpu.make_async_copy(g_hbm.at[src_v.at[0]], rows_v.at[b],
                                  sem_g.at[grp]).wait()

    def fire_scatters(j0, grp):
        for i in range(GS):
            b = grp * GS + i
            pltpu.async_copy(rows_v.at[b], acc.at[dst_v.at[j0 + i]],
                             sem_s.at[grp], add=True)

    def wait_scatters(grp):
        for i in range(GS):
            b = grp * GS + i
            pltpu.make_async_copy(rows_v.at[b], acc.at[dst_v.at[0]],
                                  sem_s.at[grp]).wait()

    for r in range(PF):
        fire_gathers(r * GS, r)
    plsc.subcore_barrier()

    @pl.loop(0, NR, step=NGRP)
    def _(r0):
        for i in range(NGRP):
            r = r0 + i
            wait_gathers(i)
            fire_scatters(r * GS, i)
            gp = (i + PF) % NGRP

            @pl.when(r + PF < NR)
            def _():
                @pl.when(r + PF >= NGRP)
                def _():
                    wait_scatters(gp)
                fire_gathers((r + PF) * GS, gp)

    for r in range(NR - NGRP, NR):
        wait_scatters(r % NGRP)

    plsc.subcore_barrier()
    pltpu.sync_copy(acc.at[pl.ds(s * RPT, RPT)],
                    out_hbm.at[c, pl.ds(s * RPT, RPT)])


NPK = N * F // 128
NPP = NP * F // 128


def _degs(pA_ref):
    pA = pA_ref[...]
    deg = pA[0, :NPK] + pA[1, :NPK] + 1.0
    inv = lax.rsqrt(deg)
    return deg, inv


def _tc_pre_body(pA_ref, x4_ref, w1bd_ref, g_ref):
    _, inv = _degs(pA_ref)
    hw = jnp.dot(x4_ref[...], w1bd_ref[...],
                 preferred_element_type=jnp.float32)
    g_ref[...] = hw * inv


def _combine(pA_ref, p_ref, b_ref):
    _, inv = _degs(pA_ref)
    p = p_ref[...]
    agg = p[0, :NPK] + p[1, :NPK]
    h = inv * agg + b_ref[...]
    return jnp.where(h > 0, h, jnp.exp(h) - 1.0), inv


def _tc_mid_body(pA_ref, p_ref, b_ref, w2bd_ref, g2_ref):
    h, inv = _combine(pA_ref, p_ref, b_ref)
    hw2 = jnp.dot(h, w2bd_ref[...], preferred_element_type=jnp.float32)
    g2_ref[...] = hw2 * inv


def _tc_post_body(pA_ref, p_ref, b_ref, wf1s_ref, bf1_ref,
                  wf2_ref, bf2_ref, out_ref):
    h, _ = _combine(pA_ref, p_ref, b_ref)
    pooled = jnp.sum(h, axis=0, keepdims=True)
    f = jnp.dot(pooled, wf1s_ref[...],
                preferred_element_type=jnp.float32) + bf1_ref[...]
    f = jnp.maximum(f, 0.0)
    o = jnp.dot(f, wf2_ref[...],
                preferred_element_type=jnp.float32) + bf2_ref[...]
    out_ref[...] = 1.0 / (1.0 + jnp.exp(-o))


_tc_pre = pl.pallas_call(
    _tc_pre_body,
    out_shape=jax.ShapeDtypeStruct((NPK, 128), jnp.float32),
)

_tc_mid = pl.pallas_call(
    _tc_mid_body,
    out_shape=jax.ShapeDtypeStruct((NPK, 128), jnp.float32),
)

_tc_post = pl.pallas_call(
    _tc_post_body,
    out_shape=jax.ShapeDtypeStruct((1, 1), jnp.float32),
)


def kernel(x, W1, b1, W2, b2, Wf1, bf1, Wf2, bf2, edge_index):
    ei = edge_index.reshape(2, NW, CPT, K)
    onesF = jnp.ones((K, F), jnp.float32)
    zerosF = jnp.zeros((RPT, F), jnp.float32)
    x4 = x.reshape(NPK, 4 * D)
    eye4 = jnp.eye(4, dtype=jnp.float32)
    W1bd = jnp.kron(eye4, W1)
    W2bd = jnp.kron(eye4, W2)
    Wf1s = jnp.concatenate([Wf1] * 4, axis=0)
    b1p = jnp.tile(b1, 4).reshape(1, 128)
    b2p = jnp.tile(b2, 4).reshape(1, 128)

    pA = _sc_degree(ei, onesF, zerosF).reshape(NC, NPP, 128)
    g1 = _tc_pre(pA, x4, W1bd)
    p1 = _sc_propagate(g1.reshape(N, F), ei, zerosF)
    g2 = _tc_mid(pA, p1.reshape(NC, NPP, 128), b1p, W2bd)
    p2 = _sc_propagate(g2.reshape(N, F), ei, zerosF)
    out = _tc_post(pA, p2.reshape(NC, NPP, 128), b2p, Wf1s,
                   bf1.reshape(1, 512), Wf2, bf2.reshape(1, 1))
    return out


# speedup vs baseline: 67.8924x; 1.0638x over previous
import functools

import jax
import jax.numpy as jnp
from jax import lax
from jax.experimental import pallas as pl
from jax.experimental.pallas import tpu as pltpu
from jax.experimental.pallas import tpu_sc as plsc

N = 10000
E = 320000
D = 128
F = 32

NC = 2
NS = 16
NW = NC * NS

K = 80
EPT = E // NW
CPT = EPT // K
NP = 10240
RPT = NP // NS

_MESH = plsc.VectorSubcoreMesh(
    core_axis_name="c", subcore_axis_name="s", num_cores=NC, num_subcores=NS)


DW8 = 8
RPP = RPT // 4


@functools.partial(
    pl.kernel,
    out_type=jax.ShapeDtypeStruct((NC, NP * F // 128, 128), jnp.float32),
    mesh=_MESH,
    scratch_types=[
        pltpu.VMEM_SHARED((NP, DW8), jnp.float32),
        pltpu.VMEM((CPT, K), jnp.int32),
        pltpu.VMEM((K, DW8), jnp.float32),
        pltpu.VMEM((RPT, DW8), jnp.float32),
        pltpu.VMEM((RPP, 128), jnp.float32),
        pltpu.SemaphoreType.DMA,
    ],
    compiler_params=pltpu.CompilerParams(use_tc_tiling_on_sc=False,
                                         needs_layout_passes=False),
)
def _sc_degree(ei_hbm, ones_hbm, zeros_hbm, out_hbm,
               acc, idx_v, ones_v, cslab, rep, sem):
    c = lax.axis_index("c")
    s = lax.axis_index("s")
    wid = c * NS + s
    pltpu.sync_copy(zeros_hbm, acc.at[pl.ds(s * RPT, RPT)])
    pltpu.sync_copy(ei_hbm.at[1, wid], idx_v)
    pltpu.sync_copy(ones_hbm, ones_v)
    plsc.subcore_barrier()

    DW = 24

    @pl.loop(0, DW)
    def _(j):
        pltpu.async_copy(ones_v, acc.at[idx_v.at[j]], sem, add=True)

    @pl.loop(DW, CPT)
    def _(j):
        pltpu.make_async_copy(ones_v, acc.at[idx_v.at[0]], sem).wait()
        pltpu.async_copy(ones_v, acc.at[idx_v.at[j]], sem, add=True)

    @pl.loop(0, DW)
    def _(j):
        pltpu.make_async_copy(ones_v, acc.at[idx_v.at[0]], sem).wait()

    plsc.subcore_barrier()

    pltpu.sync_copy(acc.at[pl.ds(s * RPT, RPT)], cslab)
    col0 = jnp.zeros((16,), jnp.int32)

    @pl.loop(0, RPP)
    def _(rp):
        for v in range(8):
            row = jnp.full((16,), 4 * rp + v // 2, jnp.int32)
            rep[rp, pl.ds(16 * v, 16)] = plsc.load_gather(cslab, [row, col0])

    pltpu.sync_copy(rep, out_hbm.at[c, pl.ds(s * RPP, RPP)])


GS = 5
NGRP = 5
NB = NGRP * GS
NR = CPT // GS
PF = 2


@functools.partial(
    pl.kernel,
    out_type=jax.ShapeDtypeStruct((NC, NP, F), jnp.float32),
    mesh=_MESH,
    scratch_types=[
        pltpu.VMEM_SHARED((NP, F), jnp.float32),
        pltpu.VMEM((CPT, K), jnp.int32),
        pltpu.VMEM((CPT, K), jnp.int32),
        pltpu.VMEM((NB, K, F), jnp.float32),
        pltpu.SemaphoreType.DMA((NGRP,)),
        pltpu.SemaphoreType.DMA((NGRP,)),
    ],
    compiler_params=pltpu.CompilerParams(use_tc_tiling_on_sc=False),
)
def _sc_propagate(g_hbm, ei_hbm, zeros_hbm, out_hbm,
                  acc, src_v, dst_v, rows_v, sem_g, sem_s):
    c = lax.axis_index("c")
    s = lax.axis_index("s")
    wid = c * NS + s

    @pl.when(c == 0)
    def _():
        @pl.when(s < NS - 1)
        def _():
            pltpu.sync_copy(g_hbm.at[pl.ds(s * RPT, RPT)],
                            acc.at[pl.ds(s * RPT, RPT)])

        @pl.when(s == NS - 1)
        def _():
            last = N - (NS - 1) * RPT
            pltpu.sync_copy(g_hbm.at[pl.ds((NS - 1) * RPT, last)],
                            acc.at[pl.ds((NS - 1) * RPT, last)])
            pltpu.sync_copy(zeros_hbm.at[pl.ds(0, NP - N)],
                            acc.at[pl.ds(N, NP - N)])

    @pl.when(c == 1)
    def _():
        pltpu.sync_copy(zeros_hbm, acc.at[pl.ds(s * RPT, RPT)])

    pltpu.sync_copy(ei_hbm.at[0, wid], src_v)
    pltpu.sync_copy(ei_hbm.at[1, wid], dst_v)

    def fire_gathers(j0, grp):
        for i in range(GS):
            b = grp * GS + i
            pltpu.async_copy(g_hbm.at[src_v.at[j0 + i]], rows_v.at[b],
                             sem_g.at[grp])

    def wait_gathers(grp):
        for i in range(GS):
            b = grp * GS + i
            pltpu.make_async_copy(g_hbm.at[src_v.at[0]], rows_v.at[b],
                                  sem_g.at[grp]).wait()

    def fire_scatters(j0, grp):
        for i in range(GS):
            b = grp * GS + i
            pltpu.async_copy(rows_v.at[b], acc.at[dst_v.at[j0 + i]],
                             sem_s.at[grp], add=True)

    def wait_scatters(grp):
        for i in range(GS):
            b = grp * GS + i
            pltpu.make_async_copy(rows_v.at[b], acc.at[dst_v.at[0]],
                                  sem_s.at[grp]).wait()

    for r in range(PF):
        fire_gathers(r * GS, r)
    plsc.subcore_barrier()

    @pl.loop(0, NR, step=NGRP)
    def _(r0):
        for i in range(NGRP):
            r = r0 + i
            wait_gathers(i)
            fire_scatters(r * GS, i)
            gp = (i + PF) % NGRP

            @pl.when(r + PF < NR)
            def _():
                @pl.when(r + PF >= NGRP)
                def _():
                    wait_scatters(gp)
                fire_gathers((r + PF) * GS, gp)

    for r in range(NR - NGRP, NR):
        wait_scatters(r % NGRP)

    plsc.subcore_barrier()
    pltpu.sync_copy(acc.at[pl.ds(s * RPT, RPT)],
                    out_hbm.at[c, pl.ds(s * RPT, RPT)])


NPK = N * F // 128
NPP = NP * F // 128


def _degs(pA_ref):
    pA = pA_ref[...]
    deg = pA[0, :NPK] + pA[1, :NPK] + 1.0
    inv = lax.rsqrt(deg)
    return deg, inv


def _tc_pre_body(pA_ref, x4_ref, w1bd_ref, g_ref):
    _, inv = _degs(pA_ref)
    hw = jnp.dot(x4_ref[...], w1bd_ref[...],
                 preferred_element_type=jnp.float32)
    g_ref[...] = hw * inv


def _combine(pA_ref, p_ref, b_ref):
    _, inv = _degs(pA_ref)
    p = p_ref[...]
    agg = p[0, :NPK] + p[1, :NPK]
    h = inv * agg + b_ref[...]
    return jnp.where(h > 0, h, jnp.exp(h) - 1.0), inv


def _tc_mid_body(pA_ref, p_ref, b_ref, w2bd_ref, g2_ref):
    h, inv = _combine(pA_ref, p_ref, b_ref)
    hw2 = jnp.dot(h, w2bd_ref[...], preferred_element_type=jnp.float32)
    g2_ref[...] = hw2 * inv


def _tc_post_body(pA_ref, p_ref, b_ref, wf1s_ref, bf1_ref,
                  wf2_ref, bf2_ref, out_ref):
    h, _ = _combine(pA_ref, p_ref, b_ref)
    pooled = jnp.sum(h, axis=0, keepdims=True)
    f = jnp.dot(pooled, wf1s_ref[...],
                preferred_element_type=jnp.float32) + bf1_ref[...]
    f = jnp.maximum(f, 0.0)
    o = jnp.dot(f, wf2_ref[...],
                preferred_element_type=jnp.float32) + bf2_ref[...]
    out_ref[...] = 1.0 / (1.0 + jnp.exp(-o))


_tc_pre = pl.pallas_call(
    _tc_pre_body,
    out_shape=jax.ShapeDtypeStruct((NPK, 128), jnp.float32),
)

_tc_mid = pl.pallas_call(
    _tc_mid_body,
    out_shape=jax.ShapeDtypeStruct((NPK, 128), jnp.float32),
)

_tc_post = pl.pallas_call(
    _tc_post_body,
    out_shape=jax.ShapeDtypeStruct((1, 1), jnp.float32),
)


def kernel(x, W1, b1, W2, b2, Wf1, bf1, Wf2, bf2, edge_index):
    ei = edge_index.reshape(2, NW, CPT, K)
    ones8 = jnp.ones((K, DW8), jnp.float32)
    zeros8 = jnp.zeros((RPT, DW8), jnp.float32)
    zerosF = jnp.zeros((RPT, F), jnp.float32)
    x4 = x.reshape(NPK, 4 * D)
    eye4 = jnp.eye(4, dtype=jnp.float32)
    W1bd = jnp.kron(eye4, W1)
    W2bd = jnp.kron(eye4, W2)
    Wf1s = jnp.concatenate([Wf1] * 4, axis=0)
    b1p = jnp.tile(b1, 4).reshape(1, 128)
    b2p = jnp.tile(b2, 4).reshape(1, 128)

    pA = _sc_degree(ei, ones8, zeros8)
    g1 = _tc_pre(pA, x4, W1bd)
    p1 = _sc_propagate(g1.reshape(N, F), ei, zerosF)
    g2 = _tc_mid(pA, p1.reshape(NC, NPP, 128), b1p, W2bd)
    p2 = _sc_propagate(g2.reshape(N, F), ei, zerosF)
    out = _tc_post(pA, p2.reshape(NC, NPP, 128), b2p, Wf1s,
                   bf1.reshape(1, 512), Wf2, bf2.reshape(1, 1))
    return out
